# baseline (speedup 1.0000x reference)
import functools

import jax
import jax.numpy as jnp
from jax import lax
from jax.experimental import pallas as pl
from jax.experimental.pallas import tpu as pltpu
from jax.experimental.pallas import tpu_sc as plsc

_NC = 2
_NS = 16
_NW = _NC * _NS
_L = 16


def _rsqrt(v):
    i = plsc.bitcast(v, jnp.int32)
    y = plsc.bitcast(jnp.int32(0x5F3759DF) - (i >> 1), jnp.float32)
    for _ in range(2):
        y = y * (jnp.float32(1.5) - jnp.float32(0.5) * v * y * y)
    return y


def _arccos(c):
    ax = jnp.abs(c)
    t = jnp.float32(1.0) - ax
    s = t * _rsqrt(jnp.maximum(t, jnp.float32(1e-30)))
    p = jnp.float32(1.5707288) + ax * (
        jnp.float32(-0.2121144)
        + ax * (jnp.float32(0.0742610) + ax * jnp.float32(-0.0187293)))
    r = s * p
    return jnp.where(c < jnp.float32(0.0), jnp.float32(3.14159265) - r, r)


def _make_sc_kernel(B, N, P):
    bpw = B // _NW

    mesh = plsc.VectorSubcoreMesh(
        core_axis_name="c", subcore_axis_name="s",
        num_cores=_NC, num_subcores=_NS)

    @functools.partial(
        pl.kernel,
        out_type=jax.ShapeDtypeStruct((B, P), jnp.float32),
        mesh=mesh,
        compiler_params=pltpu.CompilerParams(needs_layout_passes=False),
        scratch_types=[
            pltpu.VMEM((9, N), jnp.float32),
            pltpu.VMEM((9, N), jnp.float32),
            pltpu.VMEM((4, P), jnp.int32),
            pltpu.VMEM((P,), jnp.float32),
            pltpu.VMEM((P,), jnp.float32),
            pltpu.SemaphoreType.DMA,
            pltpu.SemaphoreType.DMA,
            pltpu.SemaphoreType.DMA,
            pltpu.SemaphoreType.DMA,
        ],
    )
    def sc_kernel(tri_hbm, pairs_hbm, out_hbm, xbuf0, xbuf1, pbuf, obuf0, obuf1,
                  isem0, isem1, osem0, osem1):
        wid = lax.axis_index("s") * _NC + lax.axis_index("c")
        base = wid * bpw
        last = base + bpw - 1

        pltpu.sync_copy(pairs_hbm, pbuf)

        qvecs = [jnp.full((_L,), q, jnp.int32) for q in range(9)]

        def compute(xref, oref):
            @plsc.parallel_loop(0, P, _L, unroll=16)
            def group(o):
                i0 = pbuf[0, pl.ds(o, _L)]
                i1 = pbuf[1, pl.ds(o, _L)]
                i2 = pbuf[2, pl.ds(o, _L)]
                i3 = pbuf[3, pl.ds(o, _L)]

                def cvec(ia, ib, c):
                    ga = gb = None
                    for k in range(3):
                        q = qvecs[3 * k + c]
                        la = plsc.load_gather(xref, [q, ia])
                        lb = plsc.load_gather(xref, [q, ib])
                        ga = la if ga is None else ga + la
                        gb = lb if gb is None else gb + lb
                    return ga - gb

                a0 = cvec(i0, i1, 0)
                a1 = cvec(i0, i1, 1)
                a2 = cvec(i0, i1, 2)
                b0 = cvec(i3, i2, 0)
                b1 = cvec(i3, i2, 1)
                b2 = cvec(i3, i2, 2)

                dot = a0 * b0 + a1 * b1 + a2 * b2
                n0 = a0 * a0 + a1 * a1 + a2 * a2
                n1 = b0 * b0 + b1 * b1 + b2 * b2
                r = _rsqrt(jnp.maximum(n0 * n1, jnp.float32(1e-30)))
                cosv = jnp.clip(dot * r, jnp.float32(-1.0), jnp.float32(1.0))
                oref[pl.ds(o, _L)] = _arccos(cosv)

        pltpu.make_async_copy(tri_hbm.at[:, base], xbuf0, isem0).start()
        pltpu.make_async_copy(tri_hbm.at[:, base + 1], xbuf1, isem1).start()

        def iter2(k, _):
            b = base + 2 * k

            pltpu.make_async_copy(tri_hbm.at[:, b], xbuf0, isem0).wait()

            @pl.when(k > 0)
            def _():
                pltpu.make_async_copy(obuf0, out_hbm.at[b], osem0).wait()

            compute(xbuf0, obuf0)
            pltpu.make_async_copy(obuf0, out_hbm.at[b], osem0).start()
            nxt0 = jnp.minimum(b + 2, last)
            pltpu.make_async_copy(tri_hbm.at[:, nxt0], xbuf0, isem0).start()

            pltpu.make_async_copy(tri_hbm.at[:, b + 1], xbuf1, isem1).wait()

            @pl.when(k > 0)
            def _():
                pltpu.make_async_copy(obuf1, out_hbm.at[b + 1], osem1).wait()

            compute(xbuf1, obuf1)
            pltpu.make_async_copy(obuf1, out_hbm.at[b + 1], osem1).start()
            nxt1 = jnp.minimum(b + 3, last)
            pltpu.make_async_copy(tri_hbm.at[:, nxt1], xbuf1, isem1).start()
            return 0

        lax.fori_loop(0, bpw // 2, iter2, 0)

        pltpu.make_async_copy(tri_hbm.at[:, last], xbuf0, isem0).wait()
        pltpu.make_async_copy(tri_hbm.at[:, last], xbuf1, isem1).wait()
        pltpu.make_async_copy(obuf0, out_hbm.at[last], osem0).wait()
        pltpu.make_async_copy(obuf1, out_hbm.at[last], osem1).wait()

    return sc_kernel


def kernel(triangles, bone_pairs):
    B, N, _, _ = triangles.shape
    P = bone_pairs.shape[0]
    planes = triangles.transpose(2, 3, 0, 1).reshape(9, B, N)
    pairs_t = bone_pairs.astype(jnp.int32).T
    return _make_sc_kernel(B, N, P)(planes, pairs_t)


# speedup vs baseline: 1.0618x; 1.0618x over previous
import functools

import jax
import jax.numpy as jnp
from jax import lax
from jax.experimental import pallas as pl
from jax.experimental.pallas import tpu as pltpu
from jax.experimental.pallas import tpu_sc as plsc

_NC = 2
_NS = 16
_NW = _NC * _NS
_L = 16


def _rsqrt(v):
    i = plsc.bitcast(v, jnp.int32)
    y = plsc.bitcast(jnp.int32(0x5F3759DF) - (i >> 1), jnp.float32)
    for _ in range(2):
        y = y * (jnp.float32(1.5) - jnp.float32(0.5) * v * y * y)
    return y


def _arccos(c):
    ax = jnp.abs(c)
    t = jnp.float32(1.0) - ax
    s = t * _rsqrt(jnp.maximum(t, jnp.float32(1e-30)))
    p = jnp.float32(1.5707288) + ax * (
        jnp.float32(-0.2121144)
        + ax * (jnp.float32(0.0742610) + ax * jnp.float32(-0.0187293)))
    r = s * p
    return jnp.where(c < jnp.float32(0.0), jnp.float32(3.14159265) - r, r)


def _make_sc_kernel(B, N, P):
    bpw = B // _NW

    mesh = plsc.VectorSubcoreMesh(
        core_axis_name="c", subcore_axis_name="s",
        num_cores=_NC, num_subcores=_NS)

    @functools.partial(
        pl.kernel,
        out_type=jax.ShapeDtypeStruct((B, P), jnp.float32),
        mesh=mesh,
        compiler_params=pltpu.CompilerParams(needs_layout_passes=False),
        scratch_types=[
            pltpu.VMEM((9, N), jnp.float32),
            pltpu.VMEM((9, N), jnp.float32),
            pltpu.VMEM((4, P), jnp.int32),
            pltpu.VMEM((3, N), jnp.float32),
            pltpu.VMEM((P,), jnp.float32),
            pltpu.VMEM((P,), jnp.float32),
            pltpu.SemaphoreType.DMA,
            pltpu.SemaphoreType.DMA,
            pltpu.SemaphoreType.DMA,
            pltpu.SemaphoreType.DMA,
        ],
    )
    def sc_kernel(tri_hbm, pairs_hbm, out_hbm, xbuf0, xbuf1, pbuf, sbuf, obuf0, obuf1,
                  isem0, isem1, osem0, osem1):
        wid = lax.axis_index("s") * _NC + lax.axis_index("c")
        base = wid * bpw
        last = base + bpw - 1

        pltpu.sync_copy(pairs_hbm, pbuf)

        cvecs = [jnp.full((_L,), c, jnp.int32) for c in range(3)]

        def compute(xref, oref):
            @plsc.parallel_loop(0, N, _L, unroll=8)
            def sgrp(n):
                for c in range(3):
                    sbuf[c, pl.ds(n, _L)] = (
                        xref[c, pl.ds(n, _L)]
                        + xref[3 + c, pl.ds(n, _L)]
                        + xref[6 + c, pl.ds(n, _L)])

            @plsc.parallel_loop(0, P, _L, unroll=8)
            def group(o):
                i0 = pbuf[0, pl.ds(o, _L)]
                i1 = pbuf[1, pl.ds(o, _L)]
                i2 = pbuf[2, pl.ds(o, _L)]
                i3 = pbuf[3, pl.ds(o, _L)]

                def cvec(ia, ib, c):
                    return (plsc.load_gather(sbuf, [cvecs[c], ia])
                            - plsc.load_gather(sbuf, [cvecs[c], ib]))

                a0 = cvec(i0, i1, 0)
                a1 = cvec(i0, i1, 1)
                a2 = cvec(i0, i1, 2)
                b0 = cvec(i3, i2, 0)
                b1 = cvec(i3, i2, 1)
                b2 = cvec(i3, i2, 2)

                dot = a0 * b0 + a1 * b1 + a2 * b2
                n0 = a0 * a0 + a1 * a1 + a2 * a2
                n1 = b0 * b0 + b1 * b1 + b2 * b2
                r = _rsqrt(jnp.maximum(n0 * n1, jnp.float32(1e-30)))
                cosv = jnp.clip(dot * r, jnp.float32(-1.0), jnp.float32(1.0))
                oref[pl.ds(o, _L)] = _arccos(cosv)

        pltpu.make_async_copy(tri_hbm.at[:, base], xbuf0, isem0).start()
        pltpu.make_async_copy(tri_hbm.at[:, base + 1], xbuf1, isem1).start()

        def iter2(k, _):
            b = base + 2 * k

            pltpu.make_async_copy(tri_hbm.at[:, b], xbuf0, isem0).wait()

            @pl.when(k > 0)
            def _():
                pltpu.make_async_copy(obuf0, out_hbm.at[b], osem0).wait()

            compute(xbuf0, obuf0)
            pltpu.make_async_copy(obuf0, out_hbm.at[b], osem0).start()
            nxt0 = jnp.minimum(b + 2, last)
            pltpu.make_async_copy(tri_hbm.at[:, nxt0], xbuf0, isem0).start()

            pltpu.make_async_copy(tri_hbm.at[:, b + 1], xbuf1, isem1).wait()

            @pl.when(k > 0)
            def _():
                pltpu.make_async_copy(obuf1, out_hbm.at[b + 1], osem1).wait()

            compute(xbuf1, obuf1)
            pltpu.make_async_copy(obuf1, out_hbm.at[b + 1], osem1).start()
            nxt1 = jnp.minimum(b + 3, last)
            pltpu.make_async_copy(tri_hbm.at[:, nxt1], xbuf1, isem1).start()
            return 0

        lax.fori_loop(0, bpw // 2, iter2, 0)

        pltpu.make_async_copy(tri_hbm.at[:, last], xbuf0, isem0).wait()
        pltpu.make_async_copy(tri_hbm.at[:, last], xbuf1, isem1).wait()
        pltpu.make_async_copy(obuf0, out_hbm.at[last], osem0).wait()
        pltpu.make_async_copy(obuf1, out_hbm.at[last], osem1).wait()

    return sc_kernel


def kernel(triangles, bone_pairs):
    B, N, _, _ = triangles.shape
    P = bone_pairs.shape[0]
    planes = triangles.transpose(2, 3, 0, 1).reshape(9, B, N)
    pairs_t = bone_pairs.astype(jnp.int32).T
    return _make_sc_kernel(B, N, P)(planes, pairs_t)


# speedup vs baseline: 1.7025x; 1.6034x over previous
import functools

import jax
import jax.numpy as jnp
from jax import lax
from jax.experimental import pallas as pl
from jax.experimental.pallas import tpu as pltpu
from jax.experimental.pallas import tpu_sc as plsc

_NC = 2
_NS = 16
_NW = _NC * _NS
_L = 16


def _rsqrt(v):
    i = plsc.bitcast(v, jnp.int32)
    y = plsc.bitcast(jnp.int32(0x5F3759DF) - (i >> 1), jnp.float32)
    for _ in range(2):
        y = y * (jnp.float32(1.5) - jnp.float32(0.5) * v * y * y)
    return y


def _arccos(c):
    ax = jnp.abs(c)
    t = jnp.float32(1.0) - ax
    s = t * _rsqrt(jnp.maximum(t, jnp.float32(1e-30)))
    p = jnp.float32(1.5707288) + ax * (
        jnp.float32(-0.2121144)
        + ax * (jnp.float32(0.0742610) + ax * jnp.float32(-0.0187293)))
    r = s * p
    return jnp.where(c < jnp.float32(0.0), jnp.float32(3.14159265) - r, r)


def _make_sc_kernel(B, N, P):
    bpw = B // _NW

    mesh = plsc.VectorSubcoreMesh(
        core_axis_name="c", subcore_axis_name="s",
        num_cores=_NC, num_subcores=_NS)

    @functools.partial(
        pl.kernel,
        out_type=jax.ShapeDtypeStruct((B, P), jnp.float32),
        mesh=mesh,
        compiler_params=pltpu.CompilerParams(needs_layout_passes=False),
        scratch_types=[
            pltpu.VMEM((9, N), jnp.float32),
            pltpu.VMEM((9, N), jnp.float32),
            pltpu.VMEM((4, P), jnp.int32),
            pltpu.VMEM((3, N), jnp.float32),
            pltpu.VMEM((3, N), jnp.float32),
            pltpu.VMEM((P,), jnp.float32),
            pltpu.VMEM((P,), jnp.float32),
            pltpu.SemaphoreType.DMA,
            pltpu.SemaphoreType.DMA,
            pltpu.SemaphoreType.DMA,
            pltpu.SemaphoreType.DMA,
        ],
    )
    def sc_kernel(tri_hbm, pairs_hbm, out_hbm, xbuf0, xbuf1, pbuf,
                  sbuf0, sbuf1, obuf0, obuf1, isem0, isem1, osem0, osem1):
        wid = lax.axis_index("s") * _NC + lax.axis_index("c")
        base = wid * bpw
        last = base + bpw - 1

        pltpu.sync_copy(pairs_hbm, pbuf)

        cvecs = [jnp.full((_L,), c, jnp.int32) for c in range(3)]
        xbufs = (xbuf0, xbuf1)
        sbufs = (sbuf0, sbuf1)
        obufs = (obuf0, obuf1)
        isems = (isem0, isem1)
        osems = (osem0, osem1)

        def sum_slice(xref, sref, n):
            for c in range(3):
                sref[c, pl.ds(n, _L)] = (
                    xref[c, pl.ds(n, _L)]
                    + xref[3 + c, pl.ds(n, _L)]
                    + xref[6 + c, pl.ds(n, _L)])

        def merged(xnext, snext, sprev, oref):
            @plsc.parallel_loop(0, P, _L, unroll=8)
            def group(o):
                sum_slice(xnext, snext, 2 * o)
                sum_slice(xnext, snext, 2 * o + _L)

                i0 = pbuf[0, pl.ds(o, _L)]
                i1 = pbuf[1, pl.ds(o, _L)]
                i2 = pbuf[2, pl.ds(o, _L)]
                i3 = pbuf[3, pl.ds(o, _L)]

                def cvec(ia, ib, c):
                    return (plsc.load_gather(sprev, [cvecs[c], ia])
                            - plsc.load_gather(sprev, [cvecs[c], ib]))

                a0 = cvec(i0, i1, 0)
                a1 = cvec(i0, i1, 1)
                a2 = cvec(i0, i1, 2)
                b0 = cvec(i3, i2, 0)
                b1 = cvec(i3, i2, 1)
                b2 = cvec(i3, i2, 2)

                dot = a0 * b0 + a1 * b1 + a2 * b2
                n0 = a0 * a0 + a1 * a1 + a2 * a2
                n1 = b0 * b0 + b1 * b1 + b2 * b2
                r = _rsqrt(jnp.maximum(n0 * n1, jnp.float32(1e-30)))
                cosv = jnp.clip(dot * r, jnp.float32(-1.0), jnp.float32(1.0))
                oref[pl.ds(o, _L)] = _arccos(cosv)

        pltpu.make_async_copy(tri_hbm.at[:, base], xbuf0, isem0).start()
        pltpu.make_async_copy(tri_hbm.at[:, base + 1], xbuf1, isem1).start()
        pltpu.make_async_copy(tri_hbm.at[:, base], xbuf0, isem0).wait()

        @plsc.parallel_loop(0, N, _L, unroll=8)
        def s_first(n):
            sum_slice(xbuf0, sbuf0, n)

        def step(k, p):
            b = base + k
            nxt = jnp.minimum(b + 2, last)
            pltpu.make_async_copy(tri_hbm.at[:, nxt], xbufs[p], isems[p]).start()
            pltpu.make_async_copy(tri_hbm.at[:, b + 1], xbufs[p ^ 1],
                                  isems[p ^ 1]).wait()

            @pl.when(k > 1)
            def _():
                pltpu.make_async_copy(obufs[p], out_hbm.at[b], osems[p]).wait()

            merged(xbufs[p ^ 1], sbufs[p ^ 1], sbufs[p], obufs[p])
            pltpu.make_async_copy(obufs[p], out_hbm.at[b], osems[p]).start()

        def iter2(k2, _):
            step(2 * k2, 0)
            step(2 * k2 + 1, 1)
            return 0

        lax.fori_loop(0, bpw // 2, iter2, 0)

        pltpu.make_async_copy(tri_hbm.at[:, last], xbuf1, isem1).wait()
        pltpu.make_async_copy(obuf0, out_hbm.at[last], osem0).wait()
        pltpu.make_async_copy(obuf1, out_hbm.at[last], osem1).wait()

    return sc_kernel


def kernel(triangles, bone_pairs):
    B, N, _, _ = triangles.shape
    P = bone_pairs.shape[0]
    planes = triangles.transpose(2, 3, 0, 1).reshape(9, B, N)
    pairs_t = bone_pairs.astype(jnp.int32).T
    return _make_sc_kernel(B, N, P)(planes, pairs_t)


# speedup vs baseline: 1.7915x; 1.0522x over previous
import functools

import jax
import jax.numpy as jnp
from jax import lax
from jax.experimental import pallas as pl
from jax.experimental.pallas import tpu as pltpu
from jax.experimental.pallas import tpu_sc as plsc

_NC = 2
_NS = 16
_NW = _NC * _NS
_L = 16


def _rsqrt(v):
    i = plsc.bitcast(v, jnp.int32)
    y = plsc.bitcast(jnp.int32(0x5F3759DF) - (i >> 1), jnp.float32)
    for _ in range(2):
        y = y * (jnp.float32(1.5) - jnp.float32(0.5) * v * y * y)
    return y


def _arccos(c):
    ax = jnp.abs(c)
    t = jnp.float32(1.0) - ax
    s = t * _rsqrt(jnp.maximum(t, jnp.float32(1e-30)))
    p = jnp.float32(1.5707288) + ax * (
        jnp.float32(-0.2121144)
        + ax * (jnp.float32(0.0742610) + ax * jnp.float32(-0.0187293)))
    r = s * p
    return jnp.where(c < jnp.float32(0.0), jnp.float32(3.14159265) - r, r)


def _make_sc_kernel(B, N, P):
    bpw = B // _NW

    mesh = plsc.VectorSubcoreMesh(
        core_axis_name="c", subcore_axis_name="s",
        num_cores=_NC, num_subcores=_NS)

    @functools.partial(
        pl.kernel,
        out_type=jax.ShapeDtypeStruct((B, P), jnp.float32),
        mesh=mesh,
        compiler_params=pltpu.CompilerParams(needs_layout_passes=False),
        scratch_types=[
            pltpu.VMEM((9, N), jnp.float32),
            pltpu.VMEM((9, N), jnp.float32),
            pltpu.VMEM((4, P), jnp.int32),
            pltpu.VMEM((3, N), jnp.float32),
            pltpu.VMEM((3, N), jnp.float32),
            pltpu.VMEM((P,), jnp.float32),
            pltpu.VMEM((P,), jnp.float32),
            pltpu.SemaphoreType.DMA,
            pltpu.SemaphoreType.DMA,
            pltpu.SemaphoreType.DMA,
            pltpu.SemaphoreType.DMA,
        ],
    )
    def sc_kernel(tri_hbm, pairs_hbm, out_hbm, xbuf0, xbuf1, pbuf,
                  sbuf0, sbuf1, obuf0, obuf1, isem0, isem1, osem0, osem1):
        wid = lax.axis_index("s") * _NC + lax.axis_index("c")
        base = wid * bpw
        last = base + bpw - 1

        pltpu.sync_copy(pairs_hbm, pbuf)

        cvecs = [jnp.full((_L,), c, jnp.int32) for c in range(3)]
        xbufs = (xbuf0, xbuf1)
        sbufs = (sbuf0, sbuf1)
        obufs = (obuf0, obuf1)
        isems = (isem0, isem1)
        osems = (osem0, osem1)

        def sum_slice(xref, sref, n):
            for c in range(3):
                sref[c, pl.ds(n, _L)] = (
                    xref[c, pl.ds(n, _L)]
                    + xref[3 + c, pl.ds(n, _L)]
                    + xref[6 + c, pl.ds(n, _L)])

        def merged(xnext, snext, sprev, oref):
            @plsc.parallel_loop(0, P, _L, unroll=4)
            def group(o):
                sum_slice(xnext, snext, 2 * o)
                sum_slice(xnext, snext, 2 * o + _L)

                i0 = pbuf[0, pl.ds(o, _L)]
                i1 = pbuf[1, pl.ds(o, _L)]
                i2 = pbuf[2, pl.ds(o, _L)]
                i3 = pbuf[3, pl.ds(o, _L)]

                def cvec(ia, ib, c):
                    return (plsc.load_gather(sprev, [cvecs[c], ia])
                            - plsc.load_gather(sprev, [cvecs[c], ib]))

                a0 = cvec(i0, i1, 0)
                a1 = cvec(i0, i1, 1)
                a2 = cvec(i0, i1, 2)
                b0 = cvec(i3, i2, 0)
                b1 = cvec(i3, i2, 1)
                b2 = cvec(i3, i2, 2)

                dot = a0 * b0 + a1 * b1 + a2 * b2
                n0 = a0 * a0 + a1 * a1 + a2 * a2
                n1 = b0 * b0 + b1 * b1 + b2 * b2
                r = _rsqrt(jnp.maximum(n0 * n1, jnp.float32(1e-30)))
                cosv = jnp.clip(dot * r, jnp.float32(-1.0), jnp.float32(1.0))
                oref[pl.ds(o, _L)] = _arccos(cosv)

        pltpu.make_async_copy(tri_hbm.at[:, base], xbuf0, isem0).start()
        pltpu.make_async_copy(tri_hbm.at[:, base + 1], xbuf1, isem1).start()
        pltpu.make_async_copy(tri_hbm.at[:, base], xbuf0, isem0).wait()

        @plsc.parallel_loop(0, N, _L, unroll=8)
        def s_first(n):
            sum_slice(xbuf0, sbuf0, n)

        def step(k, p):
            b = base + k
            nxt = jnp.minimum(b + 2, last)
            pltpu.make_async_copy(tri_hbm.at[:, nxt], xbufs[p], isems[p]).start()
            pltpu.make_async_copy(tri_hbm.at[:, b + 1], xbufs[p ^ 1],
                                  isems[p ^ 1]).wait()

            @pl.when(k > 1)
            def _():
                pltpu.make_async_copy(obufs[p], out_hbm.at[b], osems[p]).wait()

            merged(xbufs[p ^ 1], sbufs[p ^ 1], sbufs[p], obufs[p])
            pltpu.make_async_copy(obufs[p], out_hbm.at[b], osems[p]).start()

        def iter2(k2, _):
            step(2 * k2, 0)
            step(2 * k2 + 1, 1)
            return 0

        lax.fori_loop(0, bpw // 2, iter2, 0)

        pltpu.make_async_copy(tri_hbm.at[:, last], xbuf1, isem1).wait()
        pltpu.make_async_copy(obuf0, out_hbm.at[last], osem0).wait()
        pltpu.make_async_copy(obuf1, out_hbm.at[last], osem1).wait()

    return sc_kernel


def kernel(triangles, bone_pairs):
    B, N, _, _ = triangles.shape
    P = bone_pairs.shape[0]
    planes = triangles.transpose(2, 3, 0, 1).reshape(9, B, N)
    pairs_t = bone_pairs.astype(jnp.int32).T
    return _make_sc_kernel(B, N, P)(planes, pairs_t)


# speedup vs baseline: 1.7958x; 1.0024x over previous
import functools

import jax
import jax.numpy as jnp
from jax import lax
from jax.experimental import pallas as pl
from jax.experimental.pallas import tpu as pltpu
from jax.experimental.pallas import tpu_sc as plsc

_NC = 2
_NS = 16
_NW = _NC * _NS
_L = 16


def _rsqrt(v):
    i = plsc.bitcast(v, jnp.int32)
    y = plsc.bitcast(jnp.int32(0x5F3759DF) - (i >> 1), jnp.float32)
    for _ in range(2):
        y = y * (jnp.float32(1.5) - jnp.float32(0.5) * v * y * y)
    return y


def _arccos(c):
    ax = jnp.abs(c)
    t = jnp.float32(1.0) - ax
    s = t * _rsqrt(jnp.maximum(t, jnp.float32(1e-30)))
    p = jnp.float32(1.5707288) + ax * (
        jnp.float32(-0.2121144)
        + ax * (jnp.float32(0.0742610) + ax * jnp.float32(-0.0187293)))
    r = s * p
    return jnp.where(c < jnp.float32(0.0), jnp.float32(3.14159265) - r, r)


def _make_sc_kernel(B, N, P):
    bpw = B // _NW

    mesh = plsc.VectorSubcoreMesh(
        core_axis_name="c", subcore_axis_name="s",
        num_cores=_NC, num_subcores=_NS)

    @functools.partial(
        pl.kernel,
        out_type=jax.ShapeDtypeStruct((B, P), jnp.float32),
        mesh=mesh,
        compiler_params=pltpu.CompilerParams(needs_layout_passes=False),
        scratch_types=[
            pltpu.VMEM((9, N), jnp.float32),
            pltpu.VMEM((9, N), jnp.float32),
            pltpu.VMEM((4, P), jnp.int32),
            pltpu.VMEM((3, N), jnp.float32),
            pltpu.VMEM((3, N), jnp.float32),
            pltpu.VMEM((P,), jnp.float32),
            pltpu.VMEM((P,), jnp.float32),
            pltpu.SemaphoreType.DMA,
            pltpu.SemaphoreType.DMA,
            pltpu.SemaphoreType.DMA,
            pltpu.SemaphoreType.DMA,
        ],
    )
    def sc_kernel(tri_hbm, pairs_hbm, out_hbm, xbuf0, xbuf1, pbuf,
                  sbuf0, sbuf1, obuf0, obuf1, isem0, isem1, osem0, osem1):
        wid = lax.axis_index("s") * _NC + lax.axis_index("c")
        base = wid * bpw
        last = base + bpw - 1

        pltpu.sync_copy(pairs_hbm, pbuf)

        cvecs = [jnp.full((_L,), c, jnp.int32) for c in range(3)]
        xbufs = (xbuf0, xbuf1)
        sbufs = (sbuf0, sbuf1)
        obufs = (obuf0, obuf1)
        isems = (isem0, isem1)
        osems = (osem0, osem1)

        def sum_slice(xref, sref, n):
            for c in range(3):
                sref[c, pl.ds(n, _L)] = (
                    xref[c, pl.ds(n, _L)]
                    + xref[3 + c, pl.ds(n, _L)]
                    + xref[6 + c, pl.ds(n, _L)])

        def merged(xnext, snext, sprev, oref):
            @plsc.parallel_loop(0, P, _L, unroll=2)
            def group(o):
                sum_slice(xnext, snext, 2 * o)
                sum_slice(xnext, snext, 2 * o + _L)

                i0 = pbuf[0, pl.ds(o, _L)]
                i1 = pbuf[1, pl.ds(o, _L)]
                i2 = pbuf[2, pl.ds(o, _L)]
                i3 = pbuf[3, pl.ds(o, _L)]

                def cvec(ia, ib, c):
                    return (plsc.load_gather(sprev, [cvecs[c], ia])
                            - plsc.load_gather(sprev, [cvecs[c], ib]))

                a0 = cvec(i0, i1, 0)
                a1 = cvec(i0, i1, 1)
                a2 = cvec(i0, i1, 2)
                b0 = cvec(i3, i2, 0)
                b1 = cvec(i3, i2, 1)
                b2 = cvec(i3, i2, 2)

                dot = a0 * b0 + a1 * b1 + a2 * b2
                n0 = a0 * a0 + a1 * a1 + a2 * a2
                n1 = b0 * b0 + b1 * b1 + b2 * b2
                r = _rsqrt(jnp.maximum(n0 * n1, jnp.float32(1e-30)))
                cosv = jnp.clip(dot * r, jnp.float32(-1.0), jnp.float32(1.0))
                oref[pl.ds(o, _L)] = _arccos(cosv)

        pltpu.make_async_copy(tri_hbm.at[:, base], xbuf0, isem0).start()
        pltpu.make_async_copy(tri_hbm.at[:, base + 1], xbuf1, isem1).start()
        pltpu.make_async_copy(tri_hbm.at[:, base], xbuf0, isem0).wait()

        @plsc.parallel_loop(0, N, _L, unroll=8)
        def s_first(n):
            sum_slice(xbuf0, sbuf0, n)

        def step(k, p):
            b = base + k
            nxt = jnp.minimum(b + 2, last)
            pltpu.make_async_copy(tri_hbm.at[:, nxt], xbufs[p], isems[p]).start()
            pltpu.make_async_copy(tri_hbm.at[:, b + 1], xbufs[p ^ 1],
                                  isems[p ^ 1]).wait()

            @pl.when(k > 1)
            def _():
                pltpu.make_async_copy(obufs[p], out_hbm.at[b], osems[p]).wait()

            merged(xbufs[p ^ 1], sbufs[p ^ 1], sbufs[p], obufs[p])
            pltpu.make_async_copy(obufs[p], out_hbm.at[b], osems[p]).start()

        def iter2(k2, _):
            step(2 * k2, 0)
            step(2 * k2 + 1, 1)
            return 0

        lax.fori_loop(0, bpw // 2, iter2, 0)

        pltpu.make_async_copy(tri_hbm.at[:, last], xbuf1, isem1).wait()
        pltpu.make_async_copy(obuf0, out_hbm.at[last], osem0).wait()
        pltpu.make_async_copy(obuf1, out_hbm.at[last], osem1).wait()

    return sc_kernel


def kernel(triangles, bone_pairs):
    B, N, _, _ = triangles.shape
    P = bone_pairs.shape[0]
    planes = triangles.transpose(2, 3, 0, 1).reshape(9, B, N)
    pairs_t = bone_pairs.astype(jnp.int32).T
    return _make_sc_kernel(B, N, P)(planes, pairs_t)
